# Initial kernel scaffold; baseline (speedup 1.0000x reference)
import jax
import jax.numpy as jnp
from jax.experimental import pallas as pl


def kernel(time_intervals, embed_table):
    raise NotImplementedError("write your pallas kernel here")


# speedup vs baseline: 2.5093x; 2.5093x over previous
import jax
import jax.numpy as jnp
from jax import lax
from jax.experimental import pallas as pl
from jax.experimental.pallas import tpu as pltpu
from jax.experimental.pallas import tpu_sc as plsc

_MAX_POS = 2048
_D = 64
_B = 1024
_H = 200
_N = _B * _H

_info = plsc.get_sparse_core_info()
_NC, _NS = _info.num_cores, _info.num_subcores
_NW = _NC * _NS
_PER_W = _N // _NW
_CH = 128
_NCH = _PER_W // _CH


def _idx_body(t_ref, o_ref):
    tf = t_ref[...].astype(jnp.float32)
    tf = jnp.where(tf == 0.0, jnp.float32(1e-9), tf)
    pos = jnp.floor(100.0 * jnp.log(tf)).astype(jnp.int32)
    o_ref[...] = jnp.minimum(pos, _MAX_POS - 1)


def _gather_body(idx_hbm, table_hbm, out_hbm, idx_v, buf0, buf1, sem0, sem1):
    wid = lax.axis_index("s") * _NC + lax.axis_index("c")
    base = wid * _PER_W
    pltpu.sync_copy(idx_hbm.at[wid], idx_v)

    def gather0(c):
        pltpu.async_copy(table_hbm.at[idx_v.at[c]], buf0, sem0)

    def gather1(c):
        pltpu.async_copy(table_hbm.at[idx_v.at[c]], buf1, sem1)

    def wait0(c):
        pltpu.make_async_copy(table_hbm.at[idx_v.at[c]], buf0, sem0).wait()

    def wait1(c):
        pltpu.make_async_copy(table_hbm.at[idx_v.at[c]], buf1, sem1).wait()

    def store0(c):
        pltpu.sync_copy(buf0, out_hbm.at[pl.ds(base + c * _CH, _CH)])

    def store1(c):
        pltpu.sync_copy(buf1, out_hbm.at[pl.ds(base + c * _CH, _CH)])

    gather0(0)
    gather1(1)

    @pl.loop(0, _NCH - 2, step=2)
    def _(g):
        wait0(g)
        store0(g)
        gather0(g + 2)
        wait1(g + 1)
        store1(g + 1)
        gather1(g + 3)

    wait0(_NCH - 2)
    store0(_NCH - 2)
    wait1(_NCH - 1)
    store1(_NCH - 1)


_gather_call = pl.kernel(
    _gather_body,
    out_type=jax.ShapeDtypeStruct((_N, _D), jnp.float32),
    mesh=plsc.VectorSubcoreMesh(core_axis_name="c", subcore_axis_name="s"),
    scratch_types=[
        pltpu.VMEM((_NCH, _CH), jnp.int32),
        pltpu.VMEM((_CH, _D), jnp.float32),
        pltpu.VMEM((_CH, _D), jnp.float32),
        pltpu.SemaphoreType.DMA,
        pltpu.SemaphoreType.DMA,
    ],
    compiler_params=pltpu.CompilerParams(use_tc_tiling_on_sc=False),
)

_idx_call = pl.pallas_call(
    _idx_body,
    out_shape=jax.ShapeDtypeStruct((_B, _H), jnp.int32),
)


def kernel(time_intervals, embed_table):
    idx = _idx_call(time_intervals)
    out = _gather_call(idx.reshape(_NW, _NCH, _CH), embed_table)
    return out.reshape(_B, _H, _D)
